# baseline (speedup 1.0000x reference)
import functools

import jax
import jax.numpy as jnp
from jax import lax

_GATHER_DN = jax.lax.GatherDimensionNumbers(
    offset_dims=(), collapsed_slice_dims=(0,), start_index_map=(0,)
)


def _vgather(arr, idx):
    return jax.lax.gather(
        arr,
        idx[:, None],
        _GATHER_DN,
        slice_sizes=(1,),
        mode=jax.lax.GatherScatterMode.PROMISE_IN_BOUNDS,
    )
from jax.experimental import pallas as pl
from jax.experimental.pallas import tpu as pltpu
from jax.experimental.pallas import tpu_sc as plsc

N = 1600000
NUM_SEG = 50000
ROWS = 800
COLS = 2000
LANES = 16


def _scores_body(vt_ref, tt_ref, wt_ref, wv_ref, vs_ref, ts_ref):
    vs_ref[...] = jnp.sum(vt_ref[...] * wv_ref[...], axis=1, keepdims=True)

    @pl.when(pl.program_id(0) == 0)
    def _():
        ts_ref[...] = jnp.sum(tt_ref[...] * wt_ref[...], axis=1, keepdims=True)


def _merge_body(ax_ref, ac_ref, b_ref, out_ref, sx_acc, sc_acc):
    i = pl.program_id(0)
    sx = jnp.sum(ax_ref[...], axis=0, keepdims=True)
    sc = jnp.sum(ac_ref[...], axis=0, keepdims=True)

    @pl.when(i == 0)
    def _():
        sx_acc[...] = sx
        sc_acc[...] = sc

    @pl.when(i > 0)
    def _():
        sx_acc[...] += sx
        sc_acc[...] += sc

    @pl.when(i == pl.num_programs(0) - 1)
    def _():
        out_ref[...] = sx_acc[...] / jnp.maximum(sc_acc[...], 1.0) + b_ref[0, 0]


def kernel(type, value, batch, type_table, value_table, W, b):
    t2d = type.reshape(ROWS, COLS)
    v2d = value.reshape(ROWS, COLS)
    b2d = batch.reshape(ROWS, COLS)
    wt = W[0:1, 0:32]
    wv = W[0:1, 32:64]

    tn = type_table.shape[0]
    vn = value_table.shape[0]

    vblk = vn // 10
    vscore, tscore = pl.pallas_call(
        _scores_body,
        grid=(10,),
        in_specs=[
            pl.BlockSpec((vblk, 32), lambda i: (i, 0)),
            pl.BlockSpec((tn, 32), lambda i: (0, 0)),
            pl.BlockSpec((1, 32), lambda i: (0, 0)),
            pl.BlockSpec((1, 32), lambda i: (0, 0)),
        ],
        out_specs=[
            pl.BlockSpec((vblk, 1), lambda i: (i, 0)),
            pl.BlockSpec((tn, 1), lambda i: (0, 0)),
        ],
        out_shape=[
            jax.ShapeDtypeStruct((vn, 1), jnp.float32),
            jax.ShapeDtypeStruct((tn, 1), jnp.float32),
        ],
    )(value_table, type_table, wt, wv)
    vscore = vscore.reshape(vn)
    tscore = tscore.reshape(tn)

    info = plsc.get_sparse_core_info()
    nc, ns = info.num_cores, info.num_subcores
    nw = nc * ns
    rpw = ROWS // nw
    steps = COLS // LANES
    mesh = plsc.VectorSubcoreMesh(core_axis_name="c", subcore_axis_name="s")
    sc_params = pltpu.CompilerParams(needs_layout_passes=False)

    @functools.partial(
        pl.kernel,
        out_type=(
            jax.ShapeDtypeStruct((ROWS, COLS), jnp.int32),
            jax.ShapeDtypeStruct((nw, NUM_SEG), jnp.float32),
            jax.ShapeDtypeStruct((nw, NUM_SEG), jnp.float32),
        ),
        mesh=mesh,
        compiler_params=sc_params,
    )
    def stage_ab(t_hbm, v_hbm, b_hbm, ts_hbm, vs_hbm, x_hbm, ax_hbm, ac_hbm):
        wid = lax.axis_index("s") * nc + lax.axis_index("c")

        def phase_a(ttab, vtab, tbuf, vbuf, in_sem, out_sem):
            r0 = wid * rpw
            pltpu.sync_copy(ts_hbm, ttab.at[pl.ds(0, tn)])
            pltpu.async_copy(t_hbm.at[r0], tbuf.at[0], in_sem)
            pltpu.async_copy(v_hbm.at[r0], vbuf.at[0], in_sem)
            pltpu.sync_copy(vs_hbm, vtab)

            def row_body(j, carry):
                r = wid * rpw + j
                slot = lax.rem(j, 2)
                pltpu.make_async_copy(t_hbm.at[r], tbuf.at[slot], in_sem).wait()
                pltpu.make_async_copy(v_hbm.at[r], vbuf.at[slot], in_sem).wait()

                @pl.when(j + 1 < rpw)
                def _():
                    @pl.when(j >= 1)
                    def _():
                        pltpu.make_async_copy(
                            vbuf.at[1 - slot], x_hbm.at[r - 1], out_sem
                        ).wait()

                    pltpu.async_copy(t_hbm.at[r + 1], tbuf.at[1 - slot], in_sem)
                    pltpu.async_copy(v_hbm.at[r + 1], vbuf.at[1 - slot], in_sem)

                def step(t, c):
                    sl = pl.ds(t * LANES, LANES)
                    xs = plsc.load_gather(
                        ttab, [tbuf[slot, sl]]
                    ) + plsc.load_gather(vtab, [vbuf[slot, sl]])
                    vbuf[slot, sl] = plsc.bitcast(xs, jnp.int32)
                    return c

                lax.fori_loop(0, steps, step, 0, unroll=5)
                pltpu.async_copy(vbuf.at[slot], x_hbm.at[r], out_sem)
                return carry

            lax.fori_loop(0, rpw, row_body, 0)
            last = wid * rpw + rpw - 1
            pltpu.make_async_copy(
                vbuf.at[lax.rem(rpw - 1, 2)], x_hbm.at[last], out_sem
            ).wait()
            pltpu.make_async_copy(
                vbuf.at[lax.rem(rpw - 2, 2)], x_hbm.at[last - 1], out_sem
            ).wait()

        pl.run_scoped(
            phase_a,
            pltpu.VMEM((1024,), jnp.float32),
            pltpu.VMEM((vn,), jnp.float32),
            pltpu.VMEM((2, COLS), jnp.int32),
            pltpu.VMEM((2, COLS), jnp.int32),
            pltpu.SemaphoreType.DMA,
            pltpu.SemaphoreType.DMA,
        )

        def phase_b(accx, accc, bbuf, xbuf, in_sem):
            r0 = wid * rpw
            pltpu.async_copy(b_hbm.at[r0], bbuf.at[0], in_sem)
            pltpu.async_copy(x_hbm.at[r0], xbuf.at[0], in_sem)
            zv = jnp.zeros((LANES,), jnp.float32)

            def zbody(i, c):
                sl = pl.ds(i * LANES, LANES)
                accx[sl] = zv
                accc[sl] = zv
                return c

            lax.fori_loop(0, NUM_SEG // LANES, zbody, 0, unroll=8)

            def row_body(j, carry):
                r = wid * rpw + j
                slot = lax.rem(j, 2)
                pltpu.make_async_copy(b_hbm.at[r], bbuf.at[slot], in_sem).wait()
                pltpu.make_async_copy(x_hbm.at[r], xbuf.at[slot], in_sem).wait()

                @pl.when(j + 1 < rpw)
                def _():
                    pltpu.async_copy(b_hbm.at[r + 1], bbuf.at[1 - slot], in_sem)
                    pltpu.async_copy(x_hbm.at[r + 1], xbuf.at[1 - slot], in_sem)

                def step(t, c):
                    sl = pl.ds(t * LANES, LANES)
                    seg = bbuf[slot, sl]
                    xs = plsc.bitcast(xbuf[slot, sl], jnp.float32)
                    ii = lax.iota(jnp.int32, LANES)
                    segp = _vgather(seg, jnp.maximum(ii - 1, 0))
                    head = (ii == 0) | (seg != segp)
                    csum = plsc.cumsum(xs)
                    start = plsc.cummax(jnp.where(head, ii, 0))
                    base = jnp.where(
                        start > 0,
                        _vgather(csum, jnp.maximum(start - 1, 0)),
                        0.0,
                    )
                    segn = _vgather(seg, jnp.minimum(ii + 1, LANES - 1))
                    last = (ii == LANES - 1) | (seg != segn)
                    cnts = (ii - start + 1).astype(jnp.float32)
                    plsc.addupdate_scatter(accx, [seg], csum - base, mask=last)
                    plsc.addupdate_scatter(accc, [seg], cnts, mask=last)
                    return c

                lax.fori_loop(0, steps, step, 0, unroll=5)
                return carry

            lax.fori_loop(0, rpw, row_body, 0)
            pltpu.sync_copy(accx, ax_hbm.at[wid])
            pltpu.sync_copy(accc, ac_hbm.at[wid])

        pl.run_scoped(
            phase_b,
            pltpu.VMEM((NUM_SEG,), jnp.float32),
            pltpu.VMEM((NUM_SEG,), jnp.float32),
            pltpu.VMEM((2, COLS), jnp.int32),
            pltpu.VMEM((2, COLS), jnp.int32),
            pltpu.SemaphoreType.DMA,
        )

    _, accx, accc = stage_ab(t2d, v2d, b2d, tscore, vscore)

    wblk = 8
    out = pl.pallas_call(
        _merge_body,
        grid=(nw // wblk,),
        in_specs=[
            pl.BlockSpec((wblk, NUM_SEG), lambda i: (i, 0)),
            pl.BlockSpec((wblk, NUM_SEG), lambda i: (i, 0)),
            pl.BlockSpec((1, 1), lambda i: (0, 0)),
        ],
        out_specs=pl.BlockSpec((1, NUM_SEG), lambda i: (0, 0)),
        out_shape=jax.ShapeDtypeStruct((1, NUM_SEG), jnp.float32),
        scratch_shapes=[
            pltpu.VMEM((1, NUM_SEG), jnp.float32),
            pltpu.VMEM((1, NUM_SEG), jnp.float32),
        ],
    )(accx, accc, b.reshape(1, 1))

    return out.reshape(NUM_SEG, 1)


# speedup vs baseline: 1.0971x; 1.0971x over previous
import functools

import jax
import jax.numpy as jnp
from jax import lax

_GATHER_DN = jax.lax.GatherDimensionNumbers(
    offset_dims=(), collapsed_slice_dims=(0,), start_index_map=(0,)
)


def _vgather(arr, idx):
    return jax.lax.gather(
        arr,
        idx[:, None],
        _GATHER_DN,
        slice_sizes=(1,),
        mode=jax.lax.GatherScatterMode.PROMISE_IN_BOUNDS,
    )
from jax.experimental import pallas as pl
from jax.experimental.pallas import tpu as pltpu
from jax.experimental.pallas import tpu_sc as plsc

N = 1600000
NUM_SEG = 50000
ROWS = 160
COLS = 10000
LANES = 16


def _scores_body(vt_ref, tt_ref, wt_ref, wv_ref, vs_ref, ts_ref):
    vs_ref[...] = jnp.sum(vt_ref[...] * wv_ref[...], axis=1, keepdims=True)

    @pl.when(pl.program_id(0) == 0)
    def _():
        ts_ref[...] = jnp.sum(tt_ref[...] * wt_ref[...], axis=1, keepdims=True)


def _merge_body(ax_ref, ac_ref, b_ref, out_ref, sx_acc, sc_acc):
    i = pl.program_id(0)
    sx = jnp.sum(ax_ref[...], axis=0, keepdims=True)
    sc = jnp.sum(ac_ref[...], axis=0, keepdims=True)

    @pl.when(i == 0)
    def _():
        sx_acc[...] = sx
        sc_acc[...] = sc

    @pl.when(i > 0)
    def _():
        sx_acc[...] += sx
        sc_acc[...] += sc

    @pl.when(i == pl.num_programs(0) - 1)
    def _():
        out_ref[...] = sx_acc[...] / jnp.maximum(sc_acc[...], 1.0) + b_ref[0, 0]


def kernel(type, value, batch, type_table, value_table, W, b):
    t2d = type.reshape(ROWS, COLS)
    v2d = value.reshape(ROWS, COLS)
    b2d = batch.reshape(ROWS, COLS)
    wt = W[0:1, 0:32]
    wv = W[0:1, 32:64]

    tn = type_table.shape[0]
    vn = value_table.shape[0]

    vblk = vn // 10
    vscore, tscore = pl.pallas_call(
        _scores_body,
        grid=(10,),
        in_specs=[
            pl.BlockSpec((vblk, 32), lambda i: (i, 0)),
            pl.BlockSpec((tn, 32), lambda i: (0, 0)),
            pl.BlockSpec((1, 32), lambda i: (0, 0)),
            pl.BlockSpec((1, 32), lambda i: (0, 0)),
        ],
        out_specs=[
            pl.BlockSpec((vblk, 1), lambda i: (i, 0)),
            pl.BlockSpec((tn, 1), lambda i: (0, 0)),
        ],
        out_shape=[
            jax.ShapeDtypeStruct((vn, 1), jnp.float32),
            jax.ShapeDtypeStruct((tn, 1), jnp.float32),
        ],
    )(value_table, type_table, wt, wv)
    vscore = vscore.reshape(vn)
    tscore = tscore.reshape(tn)

    info = plsc.get_sparse_core_info()
    nc, ns = info.num_cores, info.num_subcores
    nw = nc * ns
    rpw = ROWS // nw
    steps = COLS // LANES
    mesh = plsc.VectorSubcoreMesh(core_axis_name="c", subcore_axis_name="s")
    sc_params = pltpu.CompilerParams(needs_layout_passes=False)

    @functools.partial(
        pl.kernel,
        out_type=(
            jax.ShapeDtypeStruct((ROWS, COLS), jnp.int32),
            jax.ShapeDtypeStruct((nw, NUM_SEG), jnp.float32),
            jax.ShapeDtypeStruct((nw, NUM_SEG), jnp.float32),
        ),
        mesh=mesh,
        compiler_params=sc_params,
    )
    def stage_ab(t_hbm, v_hbm, b_hbm, ts_hbm, vs_hbm, x_hbm, ax_hbm, ac_hbm):
        wid = lax.axis_index("s") * nc + lax.axis_index("c")

        def phase_a(ttab, vtab, tbuf, vbuf):
            pltpu.sync_copy(ts_hbm, ttab.at[pl.ds(0, tn)])
            pltpu.sync_copy(vs_hbm, vtab)

            def row_body(j, carry):
                r = wid * rpw + j
                pltpu.sync_copy(t_hbm.at[r], tbuf)
                pltpu.sync_copy(v_hbm.at[r], vbuf)

                def step(t, c):
                    sl = pl.ds(t * LANES, LANES)
                    xs = plsc.load_gather(ttab, [tbuf[sl]]) + plsc.load_gather(
                        vtab, [vbuf[sl]]
                    )
                    vbuf[sl] = plsc.bitcast(xs, jnp.int32)
                    return c

                lax.fori_loop(0, steps, step, 0, unroll=8)
                pltpu.sync_copy(vbuf, x_hbm.at[r])
                return carry

            lax.fori_loop(0, rpw, row_body, 0)

        pl.run_scoped(
            phase_a,
            pltpu.VMEM((1024,), jnp.float32),
            pltpu.VMEM((vn,), jnp.float32),
            pltpu.VMEM((COLS,), jnp.int32),
            pltpu.VMEM((COLS,), jnp.int32),
        )

        def phase_b(accx, accc, bbuf, xbuf):
            zv = jnp.zeros((LANES,), jnp.float32)

            def zbody(i, c):
                sl = pl.ds(i * LANES, LANES)
                accx[sl] = zv
                accc[sl] = zv
                return c

            lax.fori_loop(0, NUM_SEG // LANES, zbody, 0, unroll=8)

            def row_body(j, carry):
                r = wid * rpw + j
                pltpu.sync_copy(b_hbm.at[r], bbuf)
                pltpu.sync_copy(x_hbm.at[r], xbuf)

                def step(t, c):
                    sl = pl.ds(t * LANES, LANES)
                    seg = bbuf[sl]
                    xs = plsc.bitcast(xbuf[sl], jnp.float32)
                    ii = lax.iota(jnp.int32, LANES)
                    segp = _vgather(seg, jnp.maximum(ii - 1, 0))
                    head = (ii == 0) | (seg != segp)
                    csum = plsc.cumsum(xs)
                    start = plsc.cummax(jnp.where(head, ii, 0))
                    base = jnp.where(
                        start > 0,
                        _vgather(csum, jnp.maximum(start - 1, 0)),
                        0.0,
                    )
                    segn = _vgather(seg, jnp.minimum(ii + 1, LANES - 1))
                    last = (ii == LANES - 1) | (seg != segn)
                    cnts = (ii - start + 1).astype(jnp.float32)
                    plsc.addupdate_scatter(accx, [seg], csum - base, mask=last)
                    plsc.addupdate_scatter(accc, [seg], cnts, mask=last)
                    return c

                lax.fori_loop(0, steps, step, 0, unroll=5)
                return carry

            lax.fori_loop(0, rpw, row_body, 0)
            pltpu.sync_copy(accx, ax_hbm.at[wid])
            pltpu.sync_copy(accc, ac_hbm.at[wid])

        pl.run_scoped(
            phase_b,
            pltpu.VMEM((NUM_SEG,), jnp.float32),
            pltpu.VMEM((NUM_SEG,), jnp.float32),
            pltpu.VMEM((COLS,), jnp.int32),
            pltpu.VMEM((COLS,), jnp.int32),
        )

    _, accx, accc = stage_ab(t2d, v2d, b2d, tscore, vscore)

    wblk = 8
    out = pl.pallas_call(
        _merge_body,
        grid=(nw // wblk,),
        in_specs=[
            pl.BlockSpec((wblk, NUM_SEG), lambda i: (i, 0)),
            pl.BlockSpec((wblk, NUM_SEG), lambda i: (i, 0)),
            pl.BlockSpec((1, 1), lambda i: (0, 0)),
        ],
        out_specs=pl.BlockSpec((1, NUM_SEG), lambda i: (0, 0)),
        out_shape=jax.ShapeDtypeStruct((1, NUM_SEG), jnp.float32),
        scratch_shapes=[
            pltpu.VMEM((1, NUM_SEG), jnp.float32),
            pltpu.VMEM((1, NUM_SEG), jnp.float32),
        ],
    )(accx, accc, b.reshape(1, 1))

    return out.reshape(NUM_SEG, 1)
